# baseline (speedup 1.0000x reference)
import dataclasses
import functools

import jax
import jax.numpy as jnp
import numpy as np
from jax import lax
from jax.experimental import pallas as pl
from jax.experimental.pallas import tpu as pltpu
from jax.experimental.pallas import tpu_sc as plsc

CUTOFF = 10.0

NC = 2
NS = 16
LANES = 16


def _ssp(x):
    return jnp.maximum(x, 0.0) + jnp.log1p(jnp.exp(-jnp.abs(x))) - 0.6931471805599453


def _h_body(x_ref, w_ref, b_ref, o_ref):
    o_ref[...] = (
        jnp.dot(x_ref[...], w_ref[...], preferred_element_type=jnp.float32)
        + b_ref[...]
    )


def _compute_h(x, w_t, b, block_n):
    n, f = x.shape
    return pl.pallas_call(
        _h_body,
        grid=(n // block_n,),
        in_specs=[
            pl.BlockSpec((block_n, f), lambda i: (i, 0)),
            pl.BlockSpec((f, f), lambda i: (0, 0)),
            pl.BlockSpec((1, f), lambda i: (0, 0)),
        ],
        out_specs=pl.BlockSpec((block_n, f), lambda i: (i, 0)),
        out_shape=jax.ShapeDtypeStruct((n, f), jnp.float32),
    )(x, w_t, b)


def _filters_body(g_count, nb, lane, d_ref, w1_ref, b1_ref, w2_ref, b2_ref,
                  o_ref):
    step = CUTOFF / (g_count - 1)
    coeff = -0.5 / step**2
    f = w1_ref.shape[0]
    o_col = (lax.broadcasted_iota(jnp.int32, (g_count, 1), 0)
             .astype(jnp.float32) * step)
    for r in range(nb):
        d_row = d_ref[pl.ds(r, 1), :]
        exp_t = jnp.exp(coeff * (d_row - o_col) ** 2).astype(jnp.bfloat16)
        t1_t = (
            jax.lax.dot_general(
                w1_ref[...], exp_t, (((1,), (0,)), ((), ())),
                preferred_element_type=jnp.float32,
            )
            + b1_ref[...]
        )
        ssp_l = jnp.log(1.0 + jnp.exp(t1_t)) - 0.6931471805599453
        blk = (
            jax.lax.dot_general(
                ssp_l, w2_ref[...], (((0,), (0,)), ((), ())),
                preferred_element_type=jnp.float32,
            )
            + b2_ref[...]
        )
        u = jax.lax.bitcast_convert_type(blk, jnp.int32) + 32768
        half = f // 2
        lo_bits = (u[:, :half] >> 16) & 65535
        hi_bits = u[:, half:] & (-65536)
        o_ref[pl.ds(r * lane, lane), :] = lo_bits | hi_bits


def _compute_filters(d_t, w1_s, b1_col, w2_s, b2_row, nb):
    n_rows, lane = d_t.shape
    g = w1_s.shape[1]
    f = w1_s.shape[0]
    return pl.pallas_call(
        functools.partial(_filters_body, g, nb, lane),
        grid=(n_rows // nb,),
        in_specs=[
            pl.BlockSpec((nb, lane), lambda i: (i, 0)),
            pl.BlockSpec((f, g), lambda i: (0, 0)),
            pl.BlockSpec((f, 1), lambda i: (0, 0)),
            pl.BlockSpec((f, f), lambda i: (0, 0)),
            pl.BlockSpec((1, f), lambda i: (0, 0)),
        ],
        out_specs=pl.BlockSpec((nb * lane, f // 2), lambda i: (i, 0)),
        out_shape=jax.ShapeDtypeStruct((n_rows * lane, f // 2), jnp.int32),
    )(d_t, w1_s, b1_col, w2_s, b2_row)


def _cfconv_sc(nbr3, h, filters, zeros_nf):
    n_out, f = zeros_nf.shape
    _, n_chunks, ch = nbr3.shape
    nw = NC * NS
    rows_main = (n_out // NS) // 8 * 8
    tail = n_out - rows_main * NS
    mesh = plsc.VectorSubcoreMesh(core_axis_name="c", subcore_axis_name="s")
    cp = pltpu.CompilerParams()
    if "needs_layout_passes" in pltpu.CompilerParams.__dataclass_fields__:
        cp = dataclasses.replace(cp, needs_layout_passes=False)

    @functools.partial(
        pl.kernel,
        out_type=jax.ShapeDtypeStruct((NC, n_out, f), jnp.float32),
        mesh=mesh,
        compiler_params=cp,
        scratch_types=[
            pltpu.VMEM((2, 1, ch), jnp.int32),
            pltpu.VMEM((2, 1, ch), jnp.int32),
            pltpu.VMEM((ch, f), jnp.float32),
            pltpu.VMEM((ch, f), jnp.float32),
            pltpu.VMEM((ch, f // 2), jnp.int32),
            pltpu.VMEM_SHARED((n_out, f), jnp.float32),
            pltpu.SemaphoreType.DMA,
            pltpu.SemaphoreType.DMA,
            pltpu.SemaphoreType.DMA,
            pltpu.SemaphoreType.DMA,
            pltpu.SemaphoreType.DMA,
        ],
    )
    def sc_kernel(nbr_hbm, h_hbm, filt_hbm, zero_hbm, out_hbm,
                  idx0, idx1, rows0, rows1, filt_v, acc_shared,
                  gsem0, gsem1, fsem, ssem0, ssem1):
        cid = lax.axis_index("c")
        sid = lax.axis_index("s")
        wid = sid * NC + cid

        row0 = sid * rows_main

        def drain_copy(src_ref, dst_ref):
            pltpu.sync_copy(src_ref.at[pl.ds(row0, rows_main)],
                            dst_ref.at[pl.ds(row0, rows_main)])

            if tail > 0:
                @pl.when(sid == NS - 1)
                def _():
                    pltpu.sync_copy(src_ref.at[pl.ds(rows_main * NS, tail)],
                                    dst_ref.at[pl.ds(rows_main * NS, tail)])

        drain_copy(zero_hbm, acc_shared)
        plsc.subcore_barrier()

        def fetch(c, idx_v, rows_v, gsem, ssem, wait_pred):
            pltpu.sync_copy(nbr_hbm.at[:, pl.ds(c, 1), :], idx_v)
            if wait_pred is not None:
                @pl.when(wait_pred)
                def _():
                    pltpu.make_async_copy(rows_v,
                                          acc_shared.at[idx_v.at[1, 0]],
                                          ssem).wait()
            pltpu.async_copy(h_hbm.at[idx_v.at[0, 0]], rows_v, gsem)

        def process(c, c_nf, idx_v, rows_v, gsem, ssem):
            pltpu.make_async_copy(h_hbm.at[idx_v.at[0, 0]], rows_v, gsem).wait()
            pltpu.make_async_copy(filt_hbm.at[pl.ds(c * ch, ch)], filt_v,
                                  fsem).wait()

            @pl.loop(0, ch, unroll=2)
            def _(r):
                for m in range(0, f, 2 * LANES):
                    w = filt_v[r, pl.ds(m // 2, LANES)]
                    lo = plsc.bitcast(lax.shift_left(w, 16), jnp.float32)
                    hi = plsc.bitcast(w & jnp.int32(-65536), jnp.float32)
                    rows_v[r, pl.ds(m, LANES)] = (
                        rows_v[r, pl.ds(m, LANES)] * lo
                    )
                    rows_v[r, pl.ds(m + LANES, LANES)] = (
                        rows_v[r, pl.ds(m + LANES, LANES)] * hi
                    )

            @pl.when(c_nf < n_chunks)
            def _():
                pltpu.async_copy(filt_hbm.at[pl.ds(c_nf * ch, ch)], filt_v,
                                 fsem)

            pltpu.async_copy(rows_v, acc_shared.at[idx_v.at[1, 0]], ssem,
                             add=True)

        fetch(wid, idx0, rows0, gsem0, ssem0, wait_pred=None)
        pltpu.async_copy(filt_hbm.at[pl.ds(wid * ch, ch)], filt_v, fsem)

        @pl.loop(0, (n_chunks + 2 * nw - 1) // (2 * nw))
        def _(i):
            c_a = wid + i * 2 * nw
            c_b = c_a + nw
            c_n = c_a + 2 * nw

            @pl.when(c_b < n_chunks)
            def _():
                fetch(c_b, idx1, rows1, gsem1, ssem1, wait_pred=i > 0)

            @pl.when(c_a < n_chunks)
            def _():
                process(c_a, c_b, idx0, rows0, gsem0, ssem0)

            @pl.when(c_n < n_chunks)
            def _():
                fetch(c_n, idx0, rows0, gsem0, ssem0, wait_pred=i >= 0)

            @pl.when(c_b < n_chunks)
            def _():
                process(c_b, c_n, idx1, rows1, gsem1, ssem1)

        pltpu.make_async_copy(rows0, acc_shared.at[idx0.at[1, 0]], ssem0).wait()
        pltpu.make_async_copy(rows1, acc_shared.at[idx1.at[1, 0]], ssem1).wait()
        plsc.subcore_barrier()
        drain_copy(acc_shared, out_hbm.at[cid])

    return sc_kernel(nbr3, h, filters, zeros_nf)


def _out_body(p_ref, w1_ref, b1_ref, w2_ref, b2_ref, o_ref):
    agg = p_ref[0] + p_ref[1]
    t = _ssp(
        jnp.dot(agg, w1_ref[...], preferred_element_type=jnp.float32)
        + b1_ref[...]
    )
    o_ref[...] = (
        jnp.dot(t, w2_ref[...], preferred_element_type=jnp.float32) + b2_ref[...]
    )


def _compute_out(partials, n, w1_t, b1, w2_t, b2, block_n):
    _, _, f = partials.shape
    return pl.pallas_call(
        _out_body,
        grid=(n // block_n,),
        in_specs=[
            pl.BlockSpec((NC, block_n, f), lambda i: (0, i, 0)),
            pl.BlockSpec((f, f), lambda i: (0, 0)),
            pl.BlockSpec((1, f), lambda i: (0, 0)),
            pl.BlockSpec((f, f), lambda i: (0, 0)),
            pl.BlockSpec((1, f), lambda i: (0, 0)),
        ],
        out_specs=pl.BlockSpec((block_n, f), lambda i: (i, 0)),
        out_shape=jax.ShapeDtypeStruct((n, f), jnp.float32),
    )(partials, w1_t, b1, w2_t, b2)


def kernel(neighbour_index, neighbour_distances, node_features,
           W_lin, b_lin, W_f1, b_f1, W_f2, b_f2, W_m1, b_m1, W_m2, b_m2):
    n, f = node_features.shape
    e = neighbour_distances.shape[0]
    ch = 128
    ln2 = 0.6931471805599453

    h = _compute_h(node_features, W_lin.T, b_lin.reshape(1, f), block_n=1000)

    a = np.arange(f).reshape(-1, 2, LANES)
    perm = np.concatenate([a[:, 0, :].ravel(), a[:, 1, :].ravel()])
    w2_s = W_f2.T[:, perm]
    b2_row = b_f2[perm].reshape(1, f)
    nb = 8
    lane = 256
    rows = e // lane
    rows_pad = (rows + nb - 1) // nb * nb
    d_pad = jnp.concatenate(
        [neighbour_distances,
         jnp.zeros((rows_pad * lane - e,), jnp.float32)])
    filters = _compute_filters(
        d_pad.reshape(rows_pad, lane),
        W_f1.astype(jnp.bfloat16), b_f1.reshape(f, 1),
        w2_s, b2_row,
        nb=nb,
    )
    nbr3 = neighbour_index.reshape(2, e // ch, ch)
    zeros_nf = jnp.zeros((n, f), jnp.float32)
    partials = _cfconv_sc(nbr3, h, filters, zeros_nf)
    return _compute_out(
        partials, n, W_m1.T, b_m1.reshape(1, f), W_m2.T, b_m2.reshape(1, f),
        block_n=1000,
    )


# speedup vs baseline: 1.0024x; 1.0024x over previous
import dataclasses
import functools

import jax
import jax.numpy as jnp
import numpy as np
from jax import lax
from jax.experimental import pallas as pl
from jax.experimental.pallas import tpu as pltpu
from jax.experimental.pallas import tpu_sc as plsc

CUTOFF = 10.0

NC = 2
NS = 16
LANES = 16


def _ssp(x):
    return jnp.maximum(x, 0.0) + jnp.log1p(jnp.exp(-jnp.abs(x))) - 0.6931471805599453


def _h_body(x_ref, w_ref, b_ref, o_ref):
    o_ref[...] = (
        jnp.dot(x_ref[...], w_ref[...], preferred_element_type=jnp.float32)
        + b_ref[...]
    )


def _compute_h(x, w_t, b, block_n):
    n, f = x.shape
    return pl.pallas_call(
        _h_body,
        grid=(n // block_n,),
        in_specs=[
            pl.BlockSpec((block_n, f), lambda i: (i, 0)),
            pl.BlockSpec((f, f), lambda i: (0, 0)),
            pl.BlockSpec((1, f), lambda i: (0, 0)),
        ],
        out_specs=pl.BlockSpec((block_n, f), lambda i: (i, 0)),
        out_shape=jax.ShapeDtypeStruct((n, f), jnp.float32),
    )(x, w_t, b)


def _filters_body(g_count, nb, lane, d_ref, w1_ref, b1_ref, w2_ref, b2_ref,
                  o_ref):
    step = CUTOFF / (g_count - 1)
    coeff = -0.5 / step**2
    f = w1_ref.shape[0]
    o_col = (lax.broadcasted_iota(jnp.int32, (g_count, 1), 0)
             .astype(jnp.float32) * step)
    for r in range(nb):
        d_row = d_ref[pl.ds(r, 1), :]
        exp_t = jnp.exp(coeff * (d_row - o_col) ** 2).astype(jnp.bfloat16)
        t1_t = (
            jax.lax.dot_general(
                w1_ref[...], exp_t, (((1,), (0,)), ((), ())),
                preferred_element_type=jnp.float32,
            )
            + b1_ref[...]
        )
        ssp_l = jnp.log(1.0 + jnp.exp(t1_t)) - 0.6931471805599453
        blk = (
            jax.lax.dot_general(
                ssp_l, w2_ref[...], (((0,), (0,)), ((), ())),
                preferred_element_type=jnp.float32,
            )
            + b2_ref[...]
        )
        u = jax.lax.bitcast_convert_type(blk, jnp.int32) + 32768
        half = f // 2
        lo_bits = (u[:, :half] >> 16) & 65535
        hi_bits = u[:, half:] & (-65536)
        o_ref[pl.ds(r * lane, lane), :] = lo_bits | hi_bits


def _compute_filters(d_t, w1_s, b1_col, w2_s, b2_row, nb):
    n_rows, lane = d_t.shape
    g = w1_s.shape[1]
    f = w1_s.shape[0]
    return pl.pallas_call(
        functools.partial(_filters_body, g, nb, lane),
        grid=(n_rows // nb,),
        in_specs=[
            pl.BlockSpec((nb, lane), lambda i: (i, 0)),
            pl.BlockSpec((f, g), lambda i: (0, 0)),
            pl.BlockSpec((f, 1), lambda i: (0, 0)),
            pl.BlockSpec((f, f), lambda i: (0, 0)),
            pl.BlockSpec((1, f), lambda i: (0, 0)),
        ],
        out_specs=pl.BlockSpec((nb * lane, f // 2), lambda i: (i, 0)),
        out_shape=jax.ShapeDtypeStruct((n_rows * lane, f // 2), jnp.int32),
    )(d_t, w1_s, b1_col, w2_s, b2_row)


def _cfconv_sc(nbr3, h, filters, zeros_nf):
    n_out, f = zeros_nf.shape
    _, n_chunks, ch = nbr3.shape
    nw = NC * NS
    rows_main = (n_out // NS) // 8 * 8
    tail = n_out - rows_main * NS
    mesh = plsc.VectorSubcoreMesh(core_axis_name="c", subcore_axis_name="s")
    cp = pltpu.CompilerParams()
    if "needs_layout_passes" in pltpu.CompilerParams.__dataclass_fields__:
        cp = dataclasses.replace(cp, needs_layout_passes=False)

    @functools.partial(
        pl.kernel,
        out_type=jax.ShapeDtypeStruct((NC, n_out, f), jnp.float32),
        mesh=mesh,
        compiler_params=cp,
        scratch_types=[
            pltpu.VMEM((2, 1, ch), jnp.int32),
            pltpu.VMEM((2, 1, ch), jnp.int32),
            pltpu.VMEM((ch, f), jnp.float32),
            pltpu.VMEM((ch, f), jnp.float32),
            pltpu.VMEM((ch, f // 2), jnp.int32),
            pltpu.VMEM_SHARED((n_out, f), jnp.float32),
            pltpu.SemaphoreType.DMA,
            pltpu.SemaphoreType.DMA,
            pltpu.SemaphoreType.DMA,
            pltpu.SemaphoreType.DMA,
            pltpu.SemaphoreType.DMA,
        ],
    )
    def sc_kernel(nbr_hbm, h_hbm, filt_hbm, zero_hbm, out_hbm,
                  idx0, idx1, rows0, rows1, filt_v, acc_shared,
                  gsem0, gsem1, fsem, ssem0, ssem1):
        cid = lax.axis_index("c")
        sid = lax.axis_index("s")
        wid = sid * NC + cid

        row0 = sid * rows_main

        def drain_copy(src_ref, dst_ref):
            pltpu.sync_copy(src_ref.at[pl.ds(row0, rows_main)],
                            dst_ref.at[pl.ds(row0, rows_main)])

            if tail > 0:
                @pl.when(sid == NS - 1)
                def _():
                    pltpu.sync_copy(src_ref.at[pl.ds(rows_main * NS, tail)],
                                    dst_ref.at[pl.ds(rows_main * NS, tail)])

        drain_copy(zero_hbm, acc_shared)
        plsc.subcore_barrier()

        def fetch(c, idx_v, rows_v, gsem, ssem, wait_pred):
            pltpu.sync_copy(nbr_hbm.at[:, pl.ds(c, 1), :], idx_v)
            pltpu.async_copy(h_hbm.at[idx_v.at[0, 0]], rows_v, gsem)

        def process(c, c_nf, idx_v, rows_v, gsem, ssem):
            pltpu.make_async_copy(h_hbm.at[idx_v.at[0, 0]], rows_v, gsem).wait()
            pltpu.make_async_copy(filt_hbm.at[pl.ds(c * ch, ch)], filt_v,
                                  fsem).wait()

            @pl.loop(0, ch, unroll=2)
            def _(r):
                for m in range(0, f, 2 * LANES):
                    w = filt_v[r, pl.ds(m // 2, LANES)]
                    lo = plsc.bitcast(lax.shift_left(w, 16), jnp.float32)
                    hi = plsc.bitcast(w & jnp.int32(-65536), jnp.float32)
                    rows_v[r, pl.ds(m, LANES)] = (
                        rows_v[r, pl.ds(m, LANES)] * lo
                    )
                    rows_v[r, pl.ds(m + LANES, LANES)] = (
                        rows_v[r, pl.ds(m + LANES, LANES)] * hi
                    )

            @pl.when(c_nf < n_chunks)
            def _():
                pltpu.async_copy(filt_hbm.at[pl.ds(c_nf * ch, ch)], filt_v,
                                 fsem)

            pltpu.sync_copy(rows_v, acc_shared.at[idx_v.at[1, 0]], add=True)

        fetch(wid, idx0, rows0, gsem0, ssem0, wait_pred=None)
        pltpu.async_copy(filt_hbm.at[pl.ds(wid * ch, ch)], filt_v, fsem)

        @pl.loop(0, (n_chunks + 2 * nw - 1) // (2 * nw))
        def _(i):
            c_a = wid + i * 2 * nw
            c_b = c_a + nw
            c_n = c_a + 2 * nw

            @pl.when(c_b < n_chunks)
            def _():
                fetch(c_b, idx1, rows1, gsem1, ssem1, wait_pred=i > 0)

            @pl.when(c_a < n_chunks)
            def _():
                process(c_a, c_b, idx0, rows0, gsem0, ssem0)

            @pl.when(c_n < n_chunks)
            def _():
                fetch(c_n, idx0, rows0, gsem0, ssem0, wait_pred=i >= 0)

            @pl.when(c_b < n_chunks)
            def _():
                process(c_b, c_n, idx1, rows1, gsem1, ssem1)

        plsc.subcore_barrier()
        drain_copy(acc_shared, out_hbm.at[cid])

    return sc_kernel(nbr3, h, filters, zeros_nf)


def _out_body(p_ref, w1_ref, b1_ref, w2_ref, b2_ref, o_ref):
    agg = p_ref[0] + p_ref[1]
    t = _ssp(
        jnp.dot(agg, w1_ref[...], preferred_element_type=jnp.float32)
        + b1_ref[...]
    )
    o_ref[...] = (
        jnp.dot(t, w2_ref[...], preferred_element_type=jnp.float32) + b2_ref[...]
    )


def _compute_out(partials, n, w1_t, b1, w2_t, b2, block_n):
    _, _, f = partials.shape
    return pl.pallas_call(
        _out_body,
        grid=(n // block_n,),
        in_specs=[
            pl.BlockSpec((NC, block_n, f), lambda i: (0, i, 0)),
            pl.BlockSpec((f, f), lambda i: (0, 0)),
            pl.BlockSpec((1, f), lambda i: (0, 0)),
            pl.BlockSpec((f, f), lambda i: (0, 0)),
            pl.BlockSpec((1, f), lambda i: (0, 0)),
        ],
        out_specs=pl.BlockSpec((block_n, f), lambda i: (i, 0)),
        out_shape=jax.ShapeDtypeStruct((n, f), jnp.float32),
    )(partials, w1_t, b1, w2_t, b2)


def kernel(neighbour_index, neighbour_distances, node_features,
           W_lin, b_lin, W_f1, b_f1, W_f2, b_f2, W_m1, b_m1, W_m2, b_m2):
    n, f = node_features.shape
    e = neighbour_distances.shape[0]
    ch = 128
    ln2 = 0.6931471805599453

    h = _compute_h(node_features, W_lin.T, b_lin.reshape(1, f), block_n=1000)

    a = np.arange(f).reshape(-1, 2, LANES)
    perm = np.concatenate([a[:, 0, :].ravel(), a[:, 1, :].ravel()])
    w2_s = W_f2.T[:, perm]
    b2_row = b_f2[perm].reshape(1, f)
    nb = 8
    lane = 256
    rows = e // lane
    rows_pad = (rows + nb - 1) // nb * nb
    d_pad = jnp.concatenate(
        [neighbour_distances,
         jnp.zeros((rows_pad * lane - e,), jnp.float32)])
    filters = _compute_filters(
        d_pad.reshape(rows_pad, lane),
        W_f1.astype(jnp.bfloat16), b_f1.reshape(f, 1),
        w2_s, b2_row,
        nb=nb,
    )
    nbr3 = neighbour_index.reshape(2, e // ch, ch)
    zeros_nf = jnp.zeros((n, f), jnp.float32)
    partials = _cfconv_sc(nbr3, h, filters, zeros_nf)
    return _compute_out(
        partials, n, W_m1.T, b_m1.reshape(1, f), W_m2.T, b_m2.reshape(1, f),
        block_n=1000,
    )


# speedup vs baseline: 1.3933x; 1.3900x over previous
import dataclasses
import functools

import jax
import jax.numpy as jnp
import numpy as np
from jax import lax
from jax.experimental import pallas as pl
from jax.experimental.pallas import tpu as pltpu
from jax.experimental.pallas import tpu_sc as plsc

CUTOFF = 10.0

NC = 2
NS = 16
LANES = 16


def _ssp(x):
    return jnp.maximum(x, 0.0) + jnp.log1p(jnp.exp(-jnp.abs(x))) - 0.6931471805599453


def _h_body(x_ref, w_ref, b_ref, o_ref):
    o_ref[...] = (
        jnp.dot(x_ref[...], w_ref[...], preferred_element_type=jnp.float32)
        + b_ref[...]
    )


def _compute_h(x, w_t, b, block_n):
    n, f = x.shape
    return pl.pallas_call(
        _h_body,
        grid=(n // block_n,),
        in_specs=[
            pl.BlockSpec((block_n, f), lambda i: (i, 0)),
            pl.BlockSpec((f, f), lambda i: (0, 0)),
            pl.BlockSpec((1, f), lambda i: (0, 0)),
        ],
        out_specs=pl.BlockSpec((block_n, f), lambda i: (i, 0)),
        out_shape=jax.ShapeDtypeStruct((n, f), jnp.float32),
    )(x, w_t, b)


def _filters_body(g_count, nb, lane, d_ref, w1_ref, b1_ref, w2_ref, b2_ref,
                  o_ref):
    step = CUTOFF / (g_count - 1)
    coeff = -0.5 / step**2
    f = w1_ref.shape[0]
    o_col = (lax.broadcasted_iota(jnp.int32, (g_count, 1), 0)
             .astype(jnp.float32) * step)
    for r in range(nb):
        d_row = d_ref[pl.ds(r, 1), :]
        exp_t = jnp.exp(coeff * (d_row - o_col) ** 2).astype(jnp.bfloat16)
        t1_t = (
            jax.lax.dot_general(
                w1_ref[...], exp_t, (((1,), (0,)), ((), ())),
                preferred_element_type=jnp.float32,
            )
            + b1_ref[...]
        )
        ssp_l = jnp.log(1.0 + jnp.exp(t1_t)) - 0.6931471805599453
        blk = (
            jax.lax.dot_general(
                ssp_l, w2_ref[...], (((0,), (0,)), ((), ())),
                preferred_element_type=jnp.float32,
            )
            + b2_ref[...]
        )
        u = jax.lax.bitcast_convert_type(blk, jnp.int32) + 32768
        half = f // 2
        lo_bits = (u[:, :half] >> 16) & 65535
        hi_bits = u[:, half:] & (-65536)
        o_ref[pl.ds(r * lane, lane), :] = lo_bits | hi_bits


def _compute_filters(d_t, w1_s, b1_col, w2_s, b2_row, nb):
    n_rows, lane = d_t.shape
    g = w1_s.shape[1]
    f = w1_s.shape[0]
    return pl.pallas_call(
        functools.partial(_filters_body, g, nb, lane),
        grid=(n_rows // nb,),
        in_specs=[
            pl.BlockSpec((nb, lane), lambda i: (i, 0)),
            pl.BlockSpec((f, g), lambda i: (0, 0)),
            pl.BlockSpec((f, 1), lambda i: (0, 0)),
            pl.BlockSpec((f, f), lambda i: (0, 0)),
            pl.BlockSpec((1, f), lambda i: (0, 0)),
        ],
        out_specs=pl.BlockSpec((nb * lane, f // 2), lambda i: (i, 0)),
        out_shape=jax.ShapeDtypeStruct((n_rows * lane, f // 2), jnp.int32),
    )(d_t, w1_s, b1_col, w2_s, b2_row)


def _cfconv_sc(nbr3, h, filters, zeros_nf):
    n_out, f = zeros_nf.shape
    _, n_chunks, ch = nbr3.shape
    nw = NC * NS
    rows_main = (n_out // NS) // 8 * 8
    tail = n_out - rows_main * NS
    mesh = plsc.VectorSubcoreMesh(core_axis_name="c", subcore_axis_name="s")
    cp = pltpu.CompilerParams()
    if "needs_layout_passes" in pltpu.CompilerParams.__dataclass_fields__:
        cp = dataclasses.replace(cp, needs_layout_passes=False)

    @functools.partial(
        pl.kernel,
        out_type=jax.ShapeDtypeStruct((NC, n_out, f), jnp.float32),
        mesh=mesh,
        compiler_params=cp,
        scratch_types=[
            pltpu.VMEM((2, 1, ch), jnp.int32),
            pltpu.VMEM((2, 1, ch), jnp.int32),
            pltpu.VMEM((ch, f), jnp.float32),
            pltpu.VMEM((ch, f), jnp.float32),
            pltpu.VMEM((ch, f // 2), jnp.int32),
            pltpu.VMEM_SHARED((n_out, f), jnp.float32),
            pltpu.SemaphoreType.DMA,
            pltpu.SemaphoreType.DMA,
            pltpu.SemaphoreType.DMA,
            pltpu.SemaphoreType.DMA,
            pltpu.SemaphoreType.DMA,
        ],
    )
    def sc_kernel(nbr_hbm, h_hbm, filt_hbm, zero_hbm, out_hbm,
                  idx0, idx1, rows0, rows1, filt_v, acc_shared,
                  gsem0, gsem1, fsem, ssem0, ssem1):
        cid = lax.axis_index("c")
        sid = lax.axis_index("s")
        wid = sid * NC + cid

        row0 = sid * rows_main

        def drain_copy(src_ref, dst_ref):
            pltpu.sync_copy(src_ref.at[pl.ds(row0, rows_main)],
                            dst_ref.at[pl.ds(row0, rows_main)])

            if tail > 0:
                @pl.when(sid == NS - 1)
                def _():
                    pltpu.sync_copy(src_ref.at[pl.ds(rows_main * NS, tail)],
                                    dst_ref.at[pl.ds(rows_main * NS, tail)])

        drain_copy(zero_hbm, acc_shared)
        plsc.subcore_barrier()

        def fetch(c, idx_v, rows_v, gsem, ssem, wait_pred):
            pltpu.sync_copy(nbr_hbm.at[:, pl.ds(c, 1), :], idx_v)
            pltpu.async_copy(h_hbm.at[idx_v.at[0, 0]], rows_v, gsem)

        def process(c, c_nf, idx_v, rows_v, gsem, ssem):
            pltpu.make_async_copy(h_hbm.at[idx_v.at[0, 0]], rows_v, gsem).wait()
            pltpu.make_async_copy(filt_hbm.at[pl.ds(c * ch, ch)], filt_v,
                                  fsem).wait()

            @pl.loop(0, ch)
            def _(r):
                for m in range(0, f, 2 * LANES):
                    w = filt_v[r, pl.ds(m // 2, LANES)]
                    lo = plsc.bitcast(lax.shift_left(w, 16), jnp.float32)
                    hi = plsc.bitcast(w & jnp.int32(-65536), jnp.float32)
                    rows_v[r, pl.ds(m, LANES)] = (
                        rows_v[r, pl.ds(m, LANES)] * lo
                    )
                    rows_v[r, pl.ds(m + LANES, LANES)] = (
                        rows_v[r, pl.ds(m + LANES, LANES)] * hi
                    )

            @pl.when(c_nf < n_chunks)
            def _():
                pltpu.async_copy(filt_hbm.at[pl.ds(c_nf * ch, ch)], filt_v,
                                 fsem)

            pltpu.sync_copy(rows_v, acc_shared.at[idx_v.at[1, 0]], add=True)

        fetch(wid, idx0, rows0, gsem0, ssem0, wait_pred=None)
        pltpu.async_copy(filt_hbm.at[pl.ds(wid * ch, ch)], filt_v, fsem)

        @pl.loop(0, (n_chunks + 2 * nw - 1) // (2 * nw))
        def _(i):
            c_a = wid + i * 2 * nw
            c_b = c_a + nw
            c_n = c_a + 2 * nw

            @pl.when(c_b < n_chunks)
            def _():
                fetch(c_b, idx1, rows1, gsem1, ssem1, wait_pred=i > 0)

            @pl.when(c_a < n_chunks)
            def _():
                process(c_a, c_b, idx0, rows0, gsem0, ssem0)

            @pl.when(c_n < n_chunks)
            def _():
                fetch(c_n, idx0, rows0, gsem0, ssem0, wait_pred=i >= 0)

            @pl.when(c_b < n_chunks)
            def _():
                process(c_b, c_n, idx1, rows1, gsem1, ssem1)

        plsc.subcore_barrier()
        drain_copy(acc_shared, out_hbm.at[cid])

    return sc_kernel(nbr3, h, filters, zeros_nf)


def _out_body(p_ref, w1_ref, b1_ref, w2_ref, b2_ref, o_ref):
    agg = p_ref[0] + p_ref[1]
    t = _ssp(
        jnp.dot(agg, w1_ref[...], preferred_element_type=jnp.float32)
        + b1_ref[...]
    )
    o_ref[...] = (
        jnp.dot(t, w2_ref[...], preferred_element_type=jnp.float32) + b2_ref[...]
    )


def _compute_out(partials, n, w1_t, b1, w2_t, b2, block_n):
    _, _, f = partials.shape
    return pl.pallas_call(
        _out_body,
        grid=(n // block_n,),
        in_specs=[
            pl.BlockSpec((NC, block_n, f), lambda i: (0, i, 0)),
            pl.BlockSpec((f, f), lambda i: (0, 0)),
            pl.BlockSpec((1, f), lambda i: (0, 0)),
            pl.BlockSpec((f, f), lambda i: (0, 0)),
            pl.BlockSpec((1, f), lambda i: (0, 0)),
        ],
        out_specs=pl.BlockSpec((block_n, f), lambda i: (i, 0)),
        out_shape=jax.ShapeDtypeStruct((n, f), jnp.float32),
    )(partials, w1_t, b1, w2_t, b2)


def kernel(neighbour_index, neighbour_distances, node_features,
           W_lin, b_lin, W_f1, b_f1, W_f2, b_f2, W_m1, b_m1, W_m2, b_m2):
    n, f = node_features.shape
    e = neighbour_distances.shape[0]
    ch = 128
    ln2 = 0.6931471805599453

    h = _compute_h(node_features, W_lin.T, b_lin.reshape(1, f), block_n=1000)

    a = np.arange(f).reshape(-1, 2, LANES)
    perm = np.concatenate([a[:, 0, :].ravel(), a[:, 1, :].ravel()])
    w2_s = W_f2.T[:, perm]
    b2_row = b_f2[perm].reshape(1, f)
    nb = 8
    lane = 256
    rows = e // lane
    rows_pad = (rows + nb - 1) // nb * nb
    d_pad = jnp.concatenate(
        [neighbour_distances,
         jnp.zeros((rows_pad * lane - e,), jnp.float32)])
    filters = _compute_filters(
        d_pad.reshape(rows_pad, lane),
        W_f1.astype(jnp.bfloat16), b_f1.reshape(f, 1),
        w2_s, b2_row,
        nb=nb,
    )
    nbr3 = neighbour_index.reshape(2, e // ch, ch)
    zeros_nf = jnp.zeros((n, f), jnp.float32)
    partials = _cfconv_sc(nbr3, h, filters, zeros_nf)
    return _compute_out(
        partials, n, W_m1.T, b_m1.reshape(1, f), W_m2.T, b_m2.reshape(1, f),
        block_n=1000,
    )


# speedup vs baseline: 1.5718x; 1.1281x over previous
import dataclasses
import functools

import jax
import jax.numpy as jnp
import numpy as np
from jax import lax
from jax.experimental import pallas as pl
from jax.experimental.pallas import tpu as pltpu
from jax.experimental.pallas import tpu_sc as plsc

CUTOFF = 10.0

NC = 2
NS = 16
LANES = 16


def _ssp(x):
    return jnp.maximum(x, 0.0) + jnp.log1p(jnp.exp(-jnp.abs(x))) - 0.6931471805599453


def _h_body(x_ref, w_ref, b_ref, o_ref):
    o_ref[...] = (
        jnp.dot(x_ref[...], w_ref[...], preferred_element_type=jnp.float32)
        + b_ref[...]
    )


def _compute_h(x, w_t, b, block_n):
    n, f = x.shape
    return pl.pallas_call(
        _h_body,
        grid=(n // block_n,),
        in_specs=[
            pl.BlockSpec((block_n, f), lambda i: (i, 0)),
            pl.BlockSpec((f, f), lambda i: (0, 0)),
            pl.BlockSpec((1, f), lambda i: (0, 0)),
        ],
        out_specs=pl.BlockSpec((block_n, f), lambda i: (i, 0)),
        out_shape=jax.ShapeDtypeStruct((n, f), jnp.float32),
    )(x, w_t, b)


def _filters_body(g_count, nb, lane, d_ref, w1_ref, b1_ref, w2_ref, b2_ref,
                  o_ref):
    step = CUTOFF / (g_count - 1)
    coeff = -0.5 / step**2
    f = w1_ref.shape[0]
    o_col = (lax.broadcasted_iota(jnp.int32, (g_count, 1), 0)
             .astype(jnp.float32) * step)
    for r in range(nb):
        d_row = d_ref[pl.ds(r, 1), :]
        exp_t = jnp.exp(coeff * (d_row - o_col) ** 2).astype(jnp.bfloat16)
        t1_t = (
            jax.lax.dot_general(
                w1_ref[...], exp_t, (((1,), (0,)), ((), ())),
                preferred_element_type=jnp.float32,
            )
            + b1_ref[...]
        )
        ssp_l = jnp.log(1.0 + jnp.exp(t1_t)) - 0.6931471805599453
        blk = (
            jax.lax.dot_general(
                ssp_l, w2_ref[...], (((0,), (0,)), ((), ())),
                preferred_element_type=jnp.float32,
            )
            + b2_ref[...]
        )
        u = jax.lax.bitcast_convert_type(blk, jnp.int32) + 32768
        half = f // 2
        lo_bits = (u[:, :half] >> 16) & 65535
        hi_bits = u[:, half:] & (-65536)
        o_ref[pl.ds(r * lane, lane), :] = lo_bits | hi_bits


def _compute_filters(d_t, w1_s, b1_col, w2_s, b2_row, nb):
    n_rows, lane = d_t.shape
    g = w1_s.shape[1]
    f = w1_s.shape[0]
    return pl.pallas_call(
        functools.partial(_filters_body, g, nb, lane),
        grid=(n_rows // nb,),
        in_specs=[
            pl.BlockSpec((nb, lane), lambda i: (i, 0)),
            pl.BlockSpec((f, g), lambda i: (0, 0)),
            pl.BlockSpec((f, 1), lambda i: (0, 0)),
            pl.BlockSpec((f, f), lambda i: (0, 0)),
            pl.BlockSpec((1, f), lambda i: (0, 0)),
        ],
        out_specs=pl.BlockSpec((nb * lane, f // 2), lambda i: (i, 0)),
        out_shape=jax.ShapeDtypeStruct((n_rows * lane, f // 2), jnp.int32),
    )(d_t, w1_s, b1_col, w2_s, b2_row)


def _cfconv_sc(nbr3, h, filters, zeros_nf, c0, n_take):
    n_out, f = zeros_nf.shape
    _, _, ch = nbr3.shape
    hi_chunk = c0 + n_take
    nw = NC * NS
    rows_main = (n_out // NS) // 8 * 8
    tail = n_out - rows_main * NS
    mesh = plsc.VectorSubcoreMesh(core_axis_name="c", subcore_axis_name="s")
    cp = pltpu.CompilerParams()
    if "needs_layout_passes" in pltpu.CompilerParams.__dataclass_fields__:
        cp = dataclasses.replace(cp, needs_layout_passes=False)

    @functools.partial(
        pl.kernel,
        out_type=jax.ShapeDtypeStruct((NC, n_out, f), jnp.float32),
        mesh=mesh,
        compiler_params=cp,
        scratch_types=[
            pltpu.VMEM((2, 1, ch), jnp.int32),
            pltpu.VMEM((2, 1, ch), jnp.int32),
            pltpu.VMEM((ch, f), jnp.float32),
            pltpu.VMEM((ch, f), jnp.float32),
            pltpu.VMEM((ch, f // 2), jnp.int32),
            pltpu.VMEM_SHARED((n_out, f), jnp.float32),
            pltpu.SemaphoreType.DMA,
            pltpu.SemaphoreType.DMA,
            pltpu.SemaphoreType.DMA,
        ],
    )
    def sc_kernel(nbr_hbm, h_hbm, filt_hbm, zero_hbm, out_hbm,
                  idx0, idx1, rows0, rows1, filt_v, acc_shared,
                  gsem0, gsem1, fsem):
        cid = lax.axis_index("c")
        sid = lax.axis_index("s")
        wid = sid * NC + cid

        row0 = sid * rows_main

        def drain_copy(src_ref, dst_ref):
            pltpu.sync_copy(src_ref.at[pl.ds(row0, rows_main)],
                            dst_ref.at[pl.ds(row0, rows_main)])

            if tail > 0:
                @pl.when(sid == NS - 1)
                def _():
                    pltpu.sync_copy(src_ref.at[pl.ds(rows_main * NS, tail)],
                                    dst_ref.at[pl.ds(rows_main * NS, tail)])

        drain_copy(zero_hbm, acc_shared)
        plsc.subcore_barrier()

        def fetch(c, idx_v, rows_v, gsem):
            pltpu.sync_copy(nbr_hbm.at[:, pl.ds(c, 1), :], idx_v)
            pltpu.async_copy(h_hbm.at[idx_v.at[0, 0]], rows_v, gsem)

        def process(c, c_nf, idx_v, rows_v, gsem):
            pltpu.make_async_copy(h_hbm.at[idx_v.at[0, 0]], rows_v, gsem).wait()
            pltpu.make_async_copy(filt_hbm.at[pl.ds((c - c0) * ch, ch)],
                                  filt_v, fsem).wait()

            @pl.loop(0, ch)
            def _(r):
                for m in range(0, f, 2 * LANES):
                    w = filt_v[r, pl.ds(m // 2, LANES)]
                    lo = plsc.bitcast(lax.shift_left(w, 16), jnp.float32)
                    hi = plsc.bitcast(w & jnp.int32(-65536), jnp.float32)
                    rows_v[r, pl.ds(m, LANES)] = (
                        rows_v[r, pl.ds(m, LANES)] * lo
                    )
                    rows_v[r, pl.ds(m + LANES, LANES)] = (
                        rows_v[r, pl.ds(m + LANES, LANES)] * hi
                    )

            @pl.when(c_nf < hi_chunk)
            def _():
                pltpu.async_copy(filt_hbm.at[pl.ds((c_nf - c0) * ch, ch)],
                                 filt_v, fsem)

            pltpu.sync_copy(rows_v, acc_shared.at[idx_v.at[1, 0]], add=True)

        fetch(c0 + wid, idx0, rows0, gsem0)
        pltpu.async_copy(filt_hbm.at[pl.ds(wid * ch, ch)], filt_v, fsem)

        @pl.loop(0, (n_take + 2 * nw - 1) // (2 * nw))
        def _(i):
            c_a = c0 + wid + i * 2 * nw
            c_b = c_a + nw
            c_n = c_a + 2 * nw

            @pl.when(c_b < hi_chunk)
            def _():
                fetch(c_b, idx1, rows1, gsem1)

            @pl.when(c_a < hi_chunk)
            def _():
                process(c_a, c_b, idx0, rows0, gsem0)

            @pl.when(c_n < hi_chunk)
            def _():
                fetch(c_n, idx0, rows0, gsem0)

            @pl.when(c_b < hi_chunk)
            def _():
                process(c_b, c_n, idx1, rows1, gsem1)

        plsc.subcore_barrier()
        drain_copy(acc_shared, out_hbm.at[cid])

    return sc_kernel(nbr3, h, filters, zeros_nf)


def _out_body(p_ref, q_ref, w1_ref, b1_ref, w2_ref, b2_ref, o_ref):
    agg = (p_ref[0] + p_ref[1]) + (q_ref[0] + q_ref[1])
    t = _ssp(
        jnp.dot(agg, w1_ref[...], preferred_element_type=jnp.float32)
        + b1_ref[...]
    )
    o_ref[...] = (
        jnp.dot(t, w2_ref[...], preferred_element_type=jnp.float32) + b2_ref[...]
    )


def _compute_out(partials, partials2, n, w1_t, b1, w2_t, b2, block_n):
    _, _, f = partials.shape
    return pl.pallas_call(
        _out_body,
        grid=(n // block_n,),
        in_specs=[
            pl.BlockSpec((NC, block_n, f), lambda i: (0, i, 0)),
            pl.BlockSpec((NC, block_n, f), lambda i: (0, i, 0)),
            pl.BlockSpec((f, f), lambda i: (0, 0)),
            pl.BlockSpec((1, f), lambda i: (0, 0)),
            pl.BlockSpec((f, f), lambda i: (0, 0)),
            pl.BlockSpec((1, f), lambda i: (0, 0)),
        ],
        out_specs=pl.BlockSpec((block_n, f), lambda i: (i, 0)),
        out_shape=jax.ShapeDtypeStruct((n, f), jnp.float32),
    )(partials, partials2, w1_t, b1, w2_t, b2)


def kernel(neighbour_index, neighbour_distances, node_features,
           W_lin, b_lin, W_f1, b_f1, W_f2, b_f2, W_m1, b_m1, W_m2, b_m2):
    n, f = node_features.shape
    e = neighbour_distances.shape[0]
    ch = 128
    ln2 = 0.6931471805599453

    h = _compute_h(node_features, W_lin.T, b_lin.reshape(1, f), block_n=1000)

    a = np.arange(f).reshape(-1, 2, LANES)
    perm = np.concatenate([a[:, 0, :].ravel(), a[:, 1, :].ravel()])
    w2_s = W_f2.T[:, perm]
    b2_row = b_f2[perm].reshape(1, f)
    nb = 8
    lane = 256
    e_half = e // 2
    w1_bf = W_f1.astype(jnp.bfloat16)
    b1_col = b_f1.reshape(f, 1)

    def filters_for(d_half):
        rows = e_half // lane
        rows_pad = (rows + nb - 1) // nb * nb
        d_pad = jnp.concatenate(
            [d_half, jnp.zeros((rows_pad * lane - e_half,), jnp.float32)])
        return _compute_filters(
            d_pad.reshape(rows_pad, lane), w1_bf, b1_col, w2_s, b2_row, nb=nb)

    filters1 = filters_for(neighbour_distances[:e_half])
    filters2 = filters_for(neighbour_distances[e_half:])
    nbr3 = neighbour_index.reshape(2, e // ch, ch)
    zeros_nf = jnp.zeros((n, f), jnp.float32)
    chunks_half = e_half // ch
    partials1 = _cfconv_sc(nbr3, h, filters1, zeros_nf, 0, chunks_half)
    partials2 = _cfconv_sc(nbr3, h, filters2, zeros_nf, chunks_half,
                           chunks_half)
    return _compute_out(
        partials1, partials2, n,
        W_m1.T, b_m1.reshape(1, f), W_m2.T, b_m2.reshape(1, f),
        block_n=1000,
    )
